# Initial kernel scaffold; baseline (speedup 1.0000x reference)
import jax
import jax.numpy as jnp
from jax.experimental import pallas as pl


def kernel(x, edge_index, batch, params):
    raise NotImplementedError("write your pallas kernel here")


# speedup vs baseline: 5.7163x; 5.7163x over previous
import functools

import jax
import jax.numpy as jnp
from jax import lax
from jax.experimental import pallas as pl
from jax.experimental.pallas import tpu as pltpu
from jax.experimental.pallas import tpu_sc as plsc

N = 10000
E = 320000
D = 128
G = 128

_NW = 32
_EPW = E // _NW
_CH = 128
_NFULL = _EPW // _CH
_TAIL = _EPW - _NFULL * _CH
_NPAD = 10240
_RPT = _NPAD // 16


def _sc_agg_body(h_hbm, src_hbm, dst_hbm, out0_hbm, out1_hbm,
                 sidx, didx, rows, sidx_t, didx_t, rows_t, zbuf, acc, sem):
    c = lax.axis_index("c")
    s = lax.axis_index("s")
    wid = s * 2 + c
    base = wid * _EPW

    def zb(r, carry):
        for c8 in range(8):
            zbuf[r, pl.ds(c8 * 16, 16)] = jnp.zeros((16,), jnp.float32)
        return carry
    lax.fori_loop(0, 128, zb, None)
    for r in range(5):
        pltpu.sync_copy(zbuf, acc.at[pl.ds(s * _RPT + r * 128, 128)])
    plsc.subcore_barrier()

    def chunk(k, carry):
        off = base + k * _CH
        pltpu.sync_copy(src_hbm.at[pl.ds(off, _CH)], sidx)
        pltpu.sync_copy(dst_hbm.at[pl.ds(off, _CH)], didx)
        pltpu.async_copy(h_hbm.at[sidx], rows, sem).wait()
        pltpu.sync_copy(rows, acc.at[didx], add=True)
        return carry
    lax.fori_loop(0, _NFULL, chunk, None)

    off = base + _NFULL * _CH
    pltpu.sync_copy(src_hbm.at[pl.ds(off, _TAIL)], sidx_t)
    pltpu.sync_copy(dst_hbm.at[pl.ds(off, _TAIL)], didx_t)
    pltpu.async_copy(h_hbm.at[sidx_t], rows_t, sem).wait()
    pltpu.sync_copy(rows_t, acc.at[didx_t], add=True)

    plsc.subcore_barrier()

    @pl.when(c == 0)
    def _():
        pltpu.sync_copy(acc.at[pl.ds(s * _RPT, _RPT)],
                        out0_hbm.at[pl.ds(s * _RPT, _RPT)])

    @pl.when(c == 1)
    def _():
        pltpu.sync_copy(acc.at[pl.ds(s * _RPT, _RPT)],
                        out1_hbm.at[pl.ds(s * _RPT, _RPT)])


@functools.cache
def _get_sc_agg():
  return pl.kernel(
    _sc_agg_body,
    out_type=[jax.ShapeDtypeStruct((_NPAD, D), jnp.float32),
              jax.ShapeDtypeStruct((_NPAD, D), jnp.float32)],
    mesh=plsc.VectorSubcoreMesh(core_axis_name="c", subcore_axis_name="s",
                                num_cores=2, num_subcores=16),
    scratch_types=[
        pltpu.VMEM((_CH,), jnp.int32),
        pltpu.VMEM((_CH,), jnp.int32),
        pltpu.VMEM((_CH, D), jnp.float32),
        pltpu.VMEM((_TAIL,), jnp.int32),
        pltpu.VMEM((_TAIL,), jnp.int32),
        pltpu.VMEM((_TAIL, D), jnp.float32),
        pltpu.VMEM((128, D), jnp.float32),
        pltpu.VMEM_SHARED((_NPAD, D), jnp.float32),
        pltpu.SemaphoreType.DMA,
    ],
  )

_BN = 1000
_NB = N // _BN
_PREC = lax.Precision.DEFAULT


def _mlp_body(h_ref, a0_ref, a1_ref, w1_ref, b1_ref, w2_ref, b2_ref,
              h2_ref, sum_ref, sq_ref):
    i = pl.program_id(0)
    hin = h_ref[...] + a0_ref[...] + a1_ref[...]
    z = jnp.dot(hin, w1_ref[...], precision=_PREC,
                preferred_element_type=jnp.float32) + b1_ref[...]
    z = jnp.maximum(z, 0.0)
    h2 = jnp.dot(z, w2_ref[...], precision=_PREC,
                 preferred_element_type=jnp.float32) + b2_ref[...]
    h2 = jnp.maximum(h2, 0.0)
    h2_ref[...] = h2
    ps = jnp.sum(h2, axis=0, keepdims=True)
    pq = jnp.sum(h2 * h2, axis=0, keepdims=True)

    @pl.when(i == 0)
    def _():
        sum_ref[...] = ps
        sq_ref[...] = pq

    @pl.when(i != 0)
    def _():
        sum_ref[...] += ps
        sq_ref[...] += pq


_row_spec = pl.BlockSpec((_BN, D), lambda i: (i, 0))
_w_spec = pl.BlockSpec((D, D), lambda i: (0, 0))
_v_spec = pl.BlockSpec((1, D), lambda i: (0, 0))

_mlp = pl.pallas_call(
    _mlp_body,
    grid=(_NB,),
    in_specs=[
        _row_spec,
        pl.BlockSpec((_BN, D), lambda i: (i, 0)),
        pl.BlockSpec((_BN, D), lambda i: (i, 0)),
        _w_spec, _v_spec, _w_spec, _v_spec,
    ],
    out_specs=[_row_spec, _v_spec, _v_spec],
    out_shape=[
        jax.ShapeDtypeStruct((N, D), jnp.float32),
        jax.ShapeDtypeStruct((1, D), jnp.float32),
        jax.ShapeDtypeStruct((1, D), jnp.float32),
    ],
)


def _norm_body(h2_ref, sum_ref, sq_ref, g_ref, b_ref, out_ref):
    mean = sum_ref[...] * (1.0 / N)
    var = sq_ref[...] * (1.0 / N) - mean * mean
    inv = lax.rsqrt(var + 1e-5) * g_ref[...]
    out_ref[...] = (h2_ref[...] - mean) * inv + b_ref[...]


_norm = pl.pallas_call(
    _norm_body,
    grid=(_NB,),
    in_specs=[_row_spec, _v_spec, _v_spec, _v_spec, _v_spec],
    out_specs=_row_spec,
    out_shape=jax.ShapeDtypeStruct((N, D), jnp.float32),
)


def _pool_body(h2_ref, sum_ref, sq_ref, g_ref, b_ref, batch_ref, out_ref,
               acc, cnt):
    i = pl.program_id(0)

    @pl.when(i == 0)
    def _():
        acc[...] = jnp.zeros_like(acc)
        cnt[...] = jnp.zeros_like(cnt)

    mean = sum_ref[...] * (1.0 / N)
    var = sq_ref[...] * (1.0 / N) - mean * mean
    inv = lax.rsqrt(var + 1e-5) * g_ref[...]
    y = (h2_ref[...] - mean) * inv + b_ref[...]

    bvec = batch_ref[0, 0, :]
    oh = (bvec[:, None] == lax.broadcasted_iota(jnp.int32, (_BN, G), 1)
          ).astype(jnp.float32)
    acc[...] += lax.dot_general(oh, y, (((0,), (0,)), ((), ())),
                                precision=_PREC,
                                preferred_element_type=jnp.float32)
    cnt[...] += lax.dot_general(oh, jnp.ones((_BN, 8), jnp.float32),
                                (((0,), (0,)), ((), ())),
                                precision=_PREC,
                                preferred_element_type=jnp.float32)

    @pl.when(i == _NB - 1)
    def _():
        out_ref[...] = acc[...] / jnp.maximum(cnt[...][:, :1], 1.0)


_pool = pl.pallas_call(
    _pool_body,
    grid=(_NB,),
    in_specs=[
        _row_spec, _v_spec, _v_spec, _v_spec, _v_spec,
        pl.BlockSpec((1, 1, _BN), lambda i: (i, 0, 0)),
    ],
    out_specs=pl.BlockSpec((G, G), lambda i: (0, 0)),
    out_shape=jax.ShapeDtypeStruct((G, G), jnp.float32),
    scratch_shapes=[
        pltpu.VMEM((G, G), jnp.float32),
        pltpu.VMEM((G, 8), jnp.float32),
    ],
)


@jax.jit
def kernel(x, edge_index, batch, params):
    src = edge_index[0]
    dst = edge_index[1]
    batch3 = batch.reshape(_NB, 1, _BN)
    h = x
    for l, (W1, b1, W2, b2, gamma, beta) in enumerate(params):
        agg_a, agg_b = _get_sc_agg()(h, src, dst)
        h2, ssum, ssq = _mlp(h, agg_a, agg_b, W1, b1.reshape(1, D),
                             W2, b2.reshape(1, D))
        if l < len(params) - 1:
            h = _norm(h2, ssum, ssq, gamma.reshape(1, D), beta.reshape(1, D))
        else:
            return _pool(h2, ssum, ssq, gamma.reshape(1, D),
                         beta.reshape(1, D), batch3)
